# baseline (speedup 1.0000x reference)
import functools

import numpy as np
import jax
import jax.numpy as jnp
from jax import lax
from jax.experimental import pallas as pl
from jax.experimental.pallas import tpu as pltpu
from jax.experimental.pallas import tpu_sc as plsc

NS = 16
NV = 4
SH_DIM = 9
N_NODES = 10000
N_EDGES = 320000
N_EDGE_FEAT = 3 * NS
HIDDEN = 3 * NS
OUT_DIM = NS + 3 * NV
TP_PAD = 32

NC = 2
NSUB = 16
NW = NC * NSUB
SUB = 128
CHUNK = 1024
ROWS_PER_TILE = N_NODES // NSUB

E_HALF = N_EDGES // 2

def _mesh():
    return plsc.VectorSubcoreMesh(
        core_axis_name="c", subcore_axis_name="s", num_cores=NC, num_subcores=NSUB
    )


def _worker_chunks(body_fn, n_edges):
    n_chunks = (n_edges + CHUNK - 1) // CHUNK
    last_nb = (n_edges - (n_chunks - 1) * CHUNK) // SUB
    chunks_per_w = (n_chunks + NW - 1) // NW
    c = lax.axis_index("c")
    s = lax.axis_index("s")
    wid = s * NC + c

    def one(k, _):
        ci = wid + k * NW

        @pl.when(ci < n_chunks - 1)
        def _():
            body_fn(ci, CHUNK // SUB, ci * CHUNK)

        @pl.when(ci == n_chunks - 1)
        def _():
            body_fn(ci, last_nb, ci * CHUNK)

        return 0

    lax.fori_loop(0, chunks_per_w, one, 0)


def _gather_body(node_hbm, idx_hbm, out_hbm, idx_v, rows_v, st_v, sem, *, n_edges):
    iota16 = lax.iota(jnp.int32, 16)
    csplat = [jnp.full((16,), f, jnp.int32) for f in range(NS)]

    def chunk(ci, nb, off):
        rows = nb * SUB
        pltpu.sync_copy(idx_hbm.at[pl.ds(off, rows)], idx_v.at[pl.ds(0, rows)])
        descs = [
            pltpu.async_copy(
                node_hbm.at[idx_v.at[pl.ds(j * SUB, SUB)]],
                rows_v.at[pl.ds(j * SUB, SUB)],
                sem,
            )
            for j in range(nb)
        ]
        for d in descs:
            d.wait()

        def tpose(j, _):
            row_idx = iota16 + j * 16
            for f in range(NS):
                st_v[f, pl.ds(j * 16, 16)] = plsc.load_gather(
                    rows_v, [row_idx, csplat[f]]
                )
            return 0

        lax.fori_loop(0, nb * (SUB // 16), tpose, 0)
        fdescs = [
            pltpu.async_copy(
                st_v.at[f, pl.ds(0, rows)], out_hbm.at[f, pl.ds(off, rows)], sem
            )
            for f in range(NS)
        ]
        for d in fdescs:
            d.wait()

    _worker_chunks(chunk, n_edges)


@functools.partial(jax.jit, static_argnames="n_edges")
def _sc_gather(node_attr, edge_dst, n_edges):
    return pl.kernel(
        functools.partial(_gather_body, n_edges=n_edges),
        out_type=jax.ShapeDtypeStruct((NS, n_edges), jnp.float32),
        mesh=_mesh(),
        scratch_types=[
            pltpu.VMEM((CHUNK,), jnp.int32),
            pltpu.VMEM((CHUNK, NS), jnp.float32),
            pltpu.VMEM((NS, CHUNK), jnp.float32),
            pltpu.SemaphoreType.DMA,
        ],
        compiler_params=pltpu.CompilerParams(use_tc_tiling_on_sc=False, needs_layout_passes=False),
    )(node_attr, edge_dst)


BE = 3200
NW0 = NS * NS
NW1 = NS * NV


def _make_consts():
    R0 = np.zeros((NS, NW0), np.float32)
    R1 = np.zeros((NS, NW1), np.float32)
    for i in range(NS):
        R0[i, NS * i : NS * i + NS] = 1.0
        R1[i, NV * i : NV * i + NV] = 1.0
    P0 = np.zeros((NW0, TP_PAD), np.float32)
    for i in range(NS):
        for j in range(NS):
            P0[NS * i + j, j] = 1.0
    P1 = np.zeros((NW1, TP_PAD), np.float32)
    for i in range(NS):
        for k in range(NV):
            for m in range(3):
                P1[NV * i + k, NS + 3 * k + m] = 1.0
    S0 = np.zeros((SH_DIM, TP_PAD), np.float32)
    S0[0, :NS] = 1.0
    S1 = np.zeros((SH_DIM, TP_PAD), np.float32)
    for k in range(NV):
        for m in range(3):
            S1[1 + m, NS + 3 * k + m] = 1.0
    return R0, R1, P0, P1, S0, S1


_CONSTS = _make_consts()


def _dense_body(eaT_ref, sT_ref, shT_ref, W1_ref, b1_ref, W2_ref, b2_ref,
                R0_ref, R1_ref, P0_ref, P1_ref, S0_ref, S1_ref, TI_ref, out_ref):
    f32 = jnp.float32
    tl = (((0,), (0,)), ((), ()))
    hT = jnp.maximum(
        lax.dot_general(W1_ref[...], eaT_ref[...], tl, preferred_element_type=f32)
        + b1_ref[...],
        0.0,
    )
    wT = lax.dot_general(W2_ref[...], hT, tl, preferred_element_type=f32) + b2_ref[...]
    sT = sT_ref[...]
    X0 = lax.dot_general(R0_ref[...], sT, tl, preferred_element_type=f32) * wT[:NW0, :]
    X1 = lax.dot_general(R1_ref[...], sT, tl, preferred_element_type=f32) * wT[NW0:, :]
    t0 = lax.dot_general(P0_ref[...], X0, tl, preferred_element_type=f32)
    t1 = lax.dot_general(P1_ref[...], X1, tl, preferred_element_type=f32)
    m0 = lax.dot_general(S0_ref[...], shT_ref[...], tl, preferred_element_type=f32)
    m1 = lax.dot_general(S1_ref[...], shT_ref[...], tl, preferred_element_type=f32)
    tpT = (t0 * m0 + t1 * m1) * 0.25
    tp = lax.dot_general(tpT, TI_ref[...], tl, preferred_element_type=f32)
    col = lax.broadcasted_iota(jnp.int32, (BE, TP_PAD), 1)
    out_ref[...] = tp + jnp.where(col == OUT_DIM, 1.0, 0.0)


@functools.partial(jax.jit, static_argnames=("off", "n_edges"))
def _tc_dense(edge_attr, s_T, edge_sh, W1, b1, W2, b2, off, n_edges):
    grid = n_edges // BE
    ob = off // BE
    full = lambda shape: pl.BlockSpec(shape, lambda i: tuple(0 for _ in shape))
    eaT = edge_attr.T
    shT = edge_sh.T
    return pl.pallas_call(
        _dense_body,
        grid=(grid,),
        in_specs=[
            pl.BlockSpec((N_EDGE_FEAT, BE), lambda i: (0, i + ob)),
            pl.BlockSpec((NS, BE), lambda i: (0, i)),
            pl.BlockSpec((SH_DIM, BE), lambda i: (0, i + ob)),
            full((N_EDGE_FEAT, HIDDEN)),
            full((HIDDEN, 1)),
            full((HIDDEN, NW0 + NW1)),
            full((NW0 + NW1, 1)),
            full((NS, NW0)),
            full((NS, NW1)),
            full((NW0, TP_PAD)),
            full((NW1, TP_PAD)),
            full((SH_DIM, TP_PAD)),
            full((SH_DIM, TP_PAD)),
            full((TP_PAD, TP_PAD)),
        ],
        out_specs=pl.BlockSpec((BE, TP_PAD), lambda i: (i, 0)),
        out_shape=jax.ShapeDtypeStruct((n_edges, TP_PAD), jnp.float32),
    )(eaT, s_T, shT, W1, b1.reshape(-1, 1), W2, b2.reshape(-1, 1),
      *(jnp.asarray(c) for c in _CONSTS), jnp.eye(TP_PAD, dtype=jnp.float32))


def _scatter_body(tp_hbm, idx_hbm, zeros_hbm, out_hbm, idx2_v, data_v, acc, sem, *, n_edges):
    c = lax.axis_index("c")
    s = lax.axis_index("s")
    pltpu.sync_copy(
        zeros_hbm.at[pl.ds(s * ROWS_PER_TILE, ROWS_PER_TILE)],
        acc.at[pl.ds(s * ROWS_PER_TILE, ROWS_PER_TILE)],
    )
    plsc.subcore_barrier()

    def chunk(ci, nb, off):
        rows = nb * SUB
        d_data = pltpu.async_copy(
            tp_hbm.at[pl.ds(off, rows)], data_v.at[pl.ds(0, rows)], sem
        )
        pltpu.sync_copy(
            idx_hbm.at[pl.ds(off // SUB, nb)], idx2_v.at[pl.ds(0, nb)]
        )
        d_data.wait()
        descs = [
            pltpu.async_copy(
                data_v.at[pl.ds(j * SUB, SUB)],
                acc.at[idx2_v.at[j]],
                sem,
                add=True,
            )
            for j in range(nb)
        ]
        for d in descs:
            d.wait()

    _worker_chunks(chunk, n_edges)
    plsc.subcore_barrier()
    pltpu.sync_copy(
        acc.at[pl.ds(s * ROWS_PER_TILE, ROWS_PER_TILE)],
        out_hbm.at[c, pl.ds(s * ROWS_PER_TILE, ROWS_PER_TILE)],
    )


@functools.partial(jax.jit, static_argnames="n_edges")
def _sc_scatter(tp, edge_src, zeros_hbm, n_edges):
    return pl.kernel(
        functools.partial(_scatter_body, n_edges=n_edges),
        out_type=jax.ShapeDtypeStruct((NC, N_NODES, TP_PAD), jnp.float32),
        mesh=_mesh(),
        scratch_types=[
            pltpu.VMEM((CHUNK // SUB, SUB), jnp.int32),
            pltpu.VMEM((CHUNK, TP_PAD), jnp.float32),
            pltpu.VMEM_SHARED((N_NODES, TP_PAD), jnp.float32),
            pltpu.SemaphoreType.DMA,
        ],
        compiler_params=pltpu.CompilerParams(use_tc_tiling_on_sc=False, needs_layout_passes=False),
    )(tp, edge_src.reshape(-1, SUB), zeros_hbm)


BN = 2000


def _fin_body(p0_ref, p1_ref, out_ref):
    p = (p0_ref[0] + p0_ref[1]) + (p1_ref[0] + p1_ref[1])
    cnt = jnp.clip(p[:, OUT_DIM : OUT_DIM + 1], 1.0, None)
    out_ref[...] = p[:, :OUT_DIM] / cnt


@jax.jit
def _tc_finalize(p0, p1):
    pspec = pl.BlockSpec((NC, BN, TP_PAD), lambda i: (0, i, 0))
    return pl.pallas_call(
        _fin_body,
        grid=(N_NODES // BN,),
        in_specs=[pspec, pspec],
        out_specs=pl.BlockSpec((BN, OUT_DIM), lambda i: (i, 0)),
        out_shape=jax.ShapeDtypeStruct((N_NODES, OUT_DIM), jnp.float32),
    )(p0, p1)


def kernel(node_attr, edge_attr, edge_sh, W1, b1, W2, b2, edge_index):
    edge_src = edge_index[0]
    edge_dst = edge_index[1]
    zeros_hbm = jnp.zeros((N_NODES, TP_PAD), jnp.float32)
    sT0 = _sc_gather(node_attr, edge_dst[:E_HALF], n_edges=E_HALF)
    sT1 = _sc_gather(node_attr, edge_dst[E_HALF:], n_edges=E_HALF)
    tp0 = _tc_dense(edge_attr, sT0, edge_sh, W1, b1, W2, b2, off=0, n_edges=E_HALF)
    tp1 = _tc_dense(edge_attr, sT1, edge_sh, W1, b1, W2, b2, off=E_HALF, n_edges=E_HALF)
    p0 = _sc_scatter(tp0, edge_src[:E_HALF], zeros_hbm, n_edges=E_HALF)
    p1 = _sc_scatter(tp1, edge_src[E_HALF:], zeros_hbm, n_edges=E_HALF)
    return _tc_finalize(p0, p1)


# speedup vs baseline: 1.0104x; 1.0104x over previous
import functools

import numpy as np
import jax
import jax.numpy as jnp
from jax import lax
from jax.experimental import pallas as pl
from jax.experimental.pallas import tpu as pltpu
from jax.experimental.pallas import tpu_sc as plsc

NS = 16
NV = 4
SH_DIM = 9
N_NODES = 10000
N_EDGES = 320000
N_EDGE_FEAT = 3 * NS
HIDDEN = 3 * NS
OUT_DIM = NS + 3 * NV
TP_PAD = 32

NC = 2
NSUB = 16
NW = NC * NSUB
SUB = 128
CHUNK = 1024
ROWS_PER_TILE = N_NODES // NSUB

E_HALF = N_EDGES // 2

def _mesh():
    return plsc.VectorSubcoreMesh(
        core_axis_name="c", subcore_axis_name="s", num_cores=NC, num_subcores=NSUB
    )


def _worker_chunks(body_fn, n_edges):
    n_chunks = (n_edges + CHUNK - 1) // CHUNK
    last_nb = (n_edges - (n_chunks - 1) * CHUNK) // SUB
    chunks_per_w = (n_chunks + NW - 1) // NW
    c = lax.axis_index("c")
    s = lax.axis_index("s")
    wid = s * NC + c

    def one(k, _):
        ci = wid + k * NW

        @pl.when(ci < n_chunks - 1)
        def _():
            body_fn(ci, CHUNK // SUB, ci * CHUNK)

        @pl.when(ci == n_chunks - 1)
        def _():
            body_fn(ci, last_nb, ci * CHUNK)

        return 0

    lax.fori_loop(0, chunks_per_w, one, 0)


def _gather_body(node_hbm, idx_hbm, out_hbm, idx_v, rows_v, st_v, node_sh, sem, *, n_edges):
    csplat = [jnp.full((16,), f, jnp.int32) for f in range(NS)]

    s = lax.axis_index("s")
    pltpu.sync_copy(
        node_hbm.at[pl.ds(s * ROWS_PER_TILE, ROWS_PER_TILE)],
        node_sh.at[pl.ds(s * ROWS_PER_TILE, ROWS_PER_TILE)],
    )
    plsc.subcore_barrier()

    iota16 = lax.iota(jnp.int32, 16)

    def chunk(ci, nb, off):
        rows = nb * SUB
        pltpu.sync_copy(idx_hbm.at[pl.ds(off, rows)], idx_v.at[pl.ds(0, rows)])
        descs = [
            pltpu.async_copy(
                node_sh.at[idx_v.at[pl.ds(j * SUB, SUB)]],
                rows_v.at[pl.ds(j * SUB, SUB)],
                sem,
            )
            for j in range(nb)
        ]
        for d in descs:
            d.wait()

        def tpose(j, _):
            row_idx = iota16 + j * 16
            for f in range(NS):
                st_v[f, pl.ds(j * 16, 16)] = plsc.load_gather(
                    rows_v, [row_idx, csplat[f]]
                )
            return 0

        lax.fori_loop(0, nb * (SUB // 16), tpose, 0)
        fdescs = [
            pltpu.async_copy(
                st_v.at[f, pl.ds(0, rows)], out_hbm.at[f, pl.ds(off, rows)], sem
            )
            for f in range(NS)
        ]
        for d in fdescs:
            d.wait()

    _worker_chunks(chunk, n_edges)


@functools.partial(jax.jit, static_argnames="n_edges")
def _sc_gather(node_attr, edge_dst, n_edges):
    return pl.kernel(
        functools.partial(_gather_body, n_edges=n_edges),
        out_type=jax.ShapeDtypeStruct((NS, n_edges), jnp.float32),
        mesh=_mesh(),
        scratch_types=[
            pltpu.VMEM((CHUNK,), jnp.int32),
            pltpu.VMEM((CHUNK, NS), jnp.float32),
            pltpu.VMEM((NS, CHUNK), jnp.float32),
            pltpu.VMEM_SHARED((N_NODES, NS), jnp.float32),
            pltpu.SemaphoreType.DMA,
        ],
        compiler_params=pltpu.CompilerParams(use_tc_tiling_on_sc=False, needs_layout_passes=False),
    )(node_attr, edge_dst)


BE = 3200
NW0 = NS * NS
NW1 = NS * NV


def _make_consts():
    R0 = np.zeros((NS, NW0), np.float32)
    R1 = np.zeros((NS, NW1), np.float32)
    for i in range(NS):
        R0[i, NS * i : NS * i + NS] = 1.0
        R1[i, NV * i : NV * i + NV] = 1.0
    P0 = np.zeros((NW0, TP_PAD), np.float32)
    for i in range(NS):
        for j in range(NS):
            P0[NS * i + j, j] = 1.0
    P1 = np.zeros((NW1, TP_PAD), np.float32)
    for i in range(NS):
        for k in range(NV):
            for m in range(3):
                P1[NV * i + k, NS + 3 * k + m] = 1.0
    S0 = np.zeros((SH_DIM, TP_PAD), np.float32)
    S0[0, :NS] = 1.0
    S1 = np.zeros((SH_DIM, TP_PAD), np.float32)
    for k in range(NV):
        for m in range(3):
            S1[1 + m, NS + 3 * k + m] = 1.0
    return R0, R1, P0, P1, S0, S1


_CONSTS = _make_consts()


def _dense_body(eaT_ref, sT_ref, shT_ref, W1_ref, b1_ref, W2_ref, b2_ref,
                R0_ref, R1_ref, P0_ref, P1_ref, S0_ref, S1_ref, TI_ref, out_ref):
    f32 = jnp.float32
    tl = (((0,), (0,)), ((), ()))
    hT = jnp.maximum(
        lax.dot_general(W1_ref[...], eaT_ref[...], tl, preferred_element_type=f32)
        + b1_ref[...],
        0.0,
    )
    wT = lax.dot_general(W2_ref[...], hT, tl, preferred_element_type=f32) + b2_ref[...]
    sT = sT_ref[...]
    X0 = lax.dot_general(R0_ref[...], sT, tl, preferred_element_type=f32) * wT[:NW0, :]
    X1 = lax.dot_general(R1_ref[...], sT, tl, preferred_element_type=f32) * wT[NW0:, :]
    t0 = lax.dot_general(P0_ref[...], X0, tl, preferred_element_type=f32)
    t1 = lax.dot_general(P1_ref[...], X1, tl, preferred_element_type=f32)
    m0 = lax.dot_general(S0_ref[...], shT_ref[...], tl, preferred_element_type=f32)
    m1 = lax.dot_general(S1_ref[...], shT_ref[...], tl, preferred_element_type=f32)
    tpT = (t0 * m0 + t1 * m1) * 0.25
    tp = lax.dot_general(tpT, TI_ref[...], tl, preferred_element_type=f32)
    col = lax.broadcasted_iota(jnp.int32, (BE, TP_PAD), 1)
    out_ref[...] = tp + jnp.where(col == OUT_DIM, 1.0, 0.0)


@functools.partial(jax.jit, static_argnames=("off", "n_edges"))
def _tc_dense(edge_attr, s_T, edge_sh, W1, b1, W2, b2, off, n_edges):
    grid = n_edges // BE
    ob = off // BE
    full = lambda shape: pl.BlockSpec(shape, lambda i: tuple(0 for _ in shape))
    eaT = edge_attr.T
    shT = edge_sh.T
    return pl.pallas_call(
        _dense_body,
        grid=(grid,),
        in_specs=[
            pl.BlockSpec((N_EDGE_FEAT, BE), lambda i: (0, i + ob)),
            pl.BlockSpec((NS, BE), lambda i: (0, i)),
            pl.BlockSpec((SH_DIM, BE), lambda i: (0, i + ob)),
            full((N_EDGE_FEAT, HIDDEN)),
            full((HIDDEN, 1)),
            full((HIDDEN, NW0 + NW1)),
            full((NW0 + NW1, 1)),
            full((NS, NW0)),
            full((NS, NW1)),
            full((NW0, TP_PAD)),
            full((NW1, TP_PAD)),
            full((SH_DIM, TP_PAD)),
            full((SH_DIM, TP_PAD)),
            full((TP_PAD, TP_PAD)),
        ],
        out_specs=pl.BlockSpec((BE, TP_PAD), lambda i: (i, 0)),
        out_shape=jax.ShapeDtypeStruct((n_edges, TP_PAD), jnp.float32),
    )(eaT, s_T, shT, W1, b1.reshape(-1, 1), W2, b2.reshape(-1, 1),
      *(jnp.asarray(c) for c in _CONSTS), jnp.eye(TP_PAD, dtype=jnp.float32))


def _scatter_body(tp_hbm, idx_hbm, zeros_hbm, out_hbm, idx2_v, data_v, acc, sem, *, n_edges):
    c = lax.axis_index("c")
    s = lax.axis_index("s")
    pltpu.sync_copy(
        zeros_hbm.at[pl.ds(s * ROWS_PER_TILE, ROWS_PER_TILE)],
        acc.at[pl.ds(s * ROWS_PER_TILE, ROWS_PER_TILE)],
    )
    plsc.subcore_barrier()

    def chunk(ci, nb, off):
        rows = nb * SUB
        d_data = pltpu.async_copy(
            tp_hbm.at[pl.ds(off, rows)], data_v.at[pl.ds(0, rows)], sem
        )
        pltpu.sync_copy(
            idx_hbm.at[pl.ds(off // SUB, nb)], idx2_v.at[pl.ds(0, nb)]
        )
        d_data.wait()
        descs = [
            pltpu.async_copy(
                data_v.at[pl.ds(j * SUB, SUB)],
                acc.at[idx2_v.at[j]],
                sem,
                add=True,
            )
            for j in range(nb)
        ]
        for d in descs:
            d.wait()

    _worker_chunks(chunk, n_edges)
    plsc.subcore_barrier()
    pltpu.sync_copy(
        acc.at[pl.ds(s * ROWS_PER_TILE, ROWS_PER_TILE)],
        out_hbm.at[c, pl.ds(s * ROWS_PER_TILE, ROWS_PER_TILE)],
    )


@functools.partial(jax.jit, static_argnames="n_edges")
def _sc_scatter(tp, edge_src, zeros_hbm, n_edges):
    return pl.kernel(
        functools.partial(_scatter_body, n_edges=n_edges),
        out_type=jax.ShapeDtypeStruct((NC, N_NODES, TP_PAD), jnp.float32),
        mesh=_mesh(),
        scratch_types=[
            pltpu.VMEM((CHUNK // SUB, SUB), jnp.int32),
            pltpu.VMEM((CHUNK, TP_PAD), jnp.float32),
            pltpu.VMEM_SHARED((N_NODES, TP_PAD), jnp.float32),
            pltpu.SemaphoreType.DMA,
        ],
        compiler_params=pltpu.CompilerParams(use_tc_tiling_on_sc=False, needs_layout_passes=False),
    )(tp, edge_src.reshape(-1, SUB), zeros_hbm)


BN = 2000


def _fin_body(p0_ref, p1_ref, out_ref):
    p = (p0_ref[0] + p0_ref[1]) + (p1_ref[0] + p1_ref[1])
    cnt = jnp.clip(p[:, OUT_DIM : OUT_DIM + 1], 1.0, None)
    out_ref[...] = p[:, :OUT_DIM] / cnt


@jax.jit
def _tc_finalize(p0, p1):
    pspec = pl.BlockSpec((NC, BN, TP_PAD), lambda i: (0, i, 0))
    return pl.pallas_call(
        _fin_body,
        grid=(N_NODES // BN,),
        in_specs=[pspec, pspec],
        out_specs=pl.BlockSpec((BN, OUT_DIM), lambda i: (i, 0)),
        out_shape=jax.ShapeDtypeStruct((N_NODES, OUT_DIM), jnp.float32),
    )(p0, p1)


def kernel(node_attr, edge_attr, edge_sh, W1, b1, W2, b2, edge_index):
    edge_src = edge_index[0]
    edge_dst = edge_index[1]
    zeros_hbm = jnp.zeros((N_NODES, TP_PAD), jnp.float32)
    sT0 = _sc_gather(node_attr, edge_dst[:E_HALF], n_edges=E_HALF)
    sT1 = _sc_gather(node_attr, edge_dst[E_HALF:], n_edges=E_HALF)
    tp0 = _tc_dense(edge_attr, sT0, edge_sh, W1, b1, W2, b2, off=0, n_edges=E_HALF)
    tp1 = _tc_dense(edge_attr, sT1, edge_sh, W1, b1, W2, b2, off=E_HALF, n_edges=E_HALF)
    p0 = _sc_scatter(tp0, edge_src[:E_HALF], zeros_hbm, n_edges=E_HALF)
    p1 = _sc_scatter(tp1, edge_src[E_HALF:], zeros_hbm, n_edges=E_HALF)
    return _tc_finalize(p0, p1)
